# Initial kernel scaffold; baseline (speedup 1.0000x reference)
import jax
import jax.numpy as jnp
from jax.experimental import pallas as pl


def kernel(x, edge_index, W, b, gamma, beta):
    raise NotImplementedError("write your pallas kernel here")


# speedup vs baseline: 18.0057x; 18.0057x over previous
import functools

import jax
import jax.numpy as jnp
import numpy as np
from jax import lax
from jax.experimental import pallas as pl
from jax.experimental.pallas import tpu as pltpu
from jax.experimental.pallas import tpu_sc as plsc

NSC = 2
NSUB = 16
LANES = 16
CH = 128

F32 = jnp.float32
I32 = jnp.int32


def _cdiv(a, b):
    return (a + b - 1) // b


def _histogram(dst, n):
    e = dst.shape[0]
    npad = _cdiv(n, NSUB * LANES) * NSUB * LANES
    zslice = npad // NSUB
    epc = e // NSC
    tchunks = epc // CH
    kmax = _cdiv(tchunks, NSUB)

    mesh = plsc.VectorSubcoreMesh(core_axis_name="c", subcore_axis_name="s")

    @functools.partial(
        pl.kernel,
        mesh=mesh,
        out_type=[
            jax.ShapeDtypeStruct((npad,), F32),
            jax.ShapeDtypeStruct((npad,), F32),
        ],
        scratch_types=[
            pltpu.VMEM_SHARED((npad,), F32),
            pltpu.VMEM((kmax, CH), I32),
            pltpu.VMEM((CH,), F32),
            pltpu.VMEM((zslice,), F32),
        ],
    )
    def k(dst_hbm, p0_hbm, p1_hbm, deg_sh, idx2, ones_v, zb):
        c = lax.axis_index("c")
        s = lax.axis_index("s")

        @pl.loop(0, zslice // LANES)
        def _(i):
            zb[pl.ds(i * LANES, LANES)] = jnp.zeros((LANES,), F32)

        pltpu.sync_copy(zb, deg_sh.at[pl.ds(s * zslice, zslice)])

        @pl.loop(0, CH // LANES)
        def _(i):
            ones_v[pl.ds(i * LANES, LANES)] = jnp.ones((LANES,), F32)

        plsc.subcore_barrier()

        @pl.loop(0, kmax)
        def _(k):
            ch = k * NSUB + s

            @pl.when(ch < tchunks)
            def _():
                base = c * epc + ch * CH
                pltpu.sync_copy(dst_hbm.at[pl.ds(base, CH)], idx2.at[k])
                pltpu.sync_copy(ones_v, deg_sh.at[idx2.at[k]], add=True)

        plsc.subcore_barrier()

        @pl.when(c == 0)
        def _():
            pltpu.sync_copy(deg_sh.at[pl.ds(s * zslice, zslice)],
                            p0_hbm.at[pl.ds(s * zslice, zslice)])

        @pl.when(c == 1)
        def _():
            pltpu.sync_copy(deg_sh.at[pl.ds(s * zslice, zslice)],
                            p1_hbm.at[pl.ds(s * zslice, zslice)])

    return k(dst)


def _matmul_scale(x, w, p0, p1, rows_per_blk):
    n, din = x.shape
    dout = w.shape[1]
    half = dout // 2
    grid = n // rows_per_blk

    def body(x_ref, w_ref, p0_ref, p1_ref, y_ref, dinv_ref):
        deg = p0_ref[...] + p1_ref[...] + 1.0
        dinv = lax.rsqrt(deg)
        dinv_ref[...] = dinv
        xw = jnp.dot(x_ref[...], w_ref[...], preferred_element_type=F32)
        y = xw * dinv
        y_ref[0] = y[:, :half]
        y_ref[1] = y[:, half:]

    return pl.pallas_call(
        body,
        grid=(grid,),
        in_specs=[
            pl.BlockSpec((rows_per_blk, din), lambda i: (i, 0)),
            pl.BlockSpec((din, dout), lambda i: (0, 0)),
            pl.BlockSpec((rows_per_blk, 1), lambda i: (i, 0)),
            pl.BlockSpec((rows_per_blk, 1), lambda i: (i, 0)),
        ],
        out_specs=[
            pl.BlockSpec((NSC, rows_per_blk, half), lambda i: (0, i, 0)),
            pl.BlockSpec((rows_per_blk, 1), lambda i: (i, 0)),
        ],
        out_shape=[
            jax.ShapeDtypeStruct((NSC, n, half), F32),
            jax.ShapeDtypeStruct((n, 1), F32),
        ],
    )(x, w, p0, p1)


def _gather_scatter(y3, src, dst, n):
    e = src.shape[0]
    half = y3.shape[2]
    tchunks = e // CH
    kmax = _cdiv(tchunks, NSUB)
    rows_pw = (n // NSUB) // 8 * 8
    rem = n - (NSUB - 1) * rows_pw - rows_pw

    mesh = plsc.VectorSubcoreMesh(core_axis_name="c", subcore_axis_name="s")

    nbuf = 2

    @functools.partial(
        pl.kernel,
        mesh=mesh,
        out_type=jax.ShapeDtypeStruct((NSC, n, half), F32),
        scratch_types=[
            pltpu.VMEM_SHARED((n, half), F32),
            pltpu.VMEM((nbuf, CH), I32),
            pltpu.VMEM((nbuf, CH), I32),
            pltpu.VMEM((nbuf, CH, half), F32),
            pltpu.SemaphoreType.DMA,
            pltpu.SemaphoreType.DMA,
        ],
    )
    def k(y_hbm, src_hbm, dst_hbm, s_hbm, acc_sh, srcb, dstb, gbuf, sem0, sem1):
        c = lax.axis_index("c")
        s = lax.axis_index("s")
        yc = y_hbm.at[c]
        sems = (sem0, sem1)

        def start_gather(trip, b):
            @pl.when(trip * NSUB + s < tchunks)
            def _():
                base = (trip * NSUB + s) * CH
                pltpu.sync_copy(src_hbm.at[pl.ds(base, CH)], srcb.at[b])
                pltpu.async_copy(yc.at[srcb.at[b]], gbuf.at[b], sems[b])

        def finish_trip(trip, b):
            @pl.when(trip * NSUB + s < tchunks)
            def _():
                base = (trip * NSUB + s) * CH
                pltpu.make_async_copy(yc.at[srcb.at[b]], gbuf.at[b],
                                      sems[b]).wait()
                pltpu.sync_copy(dst_hbm.at[pl.ds(base, CH)], dstb.at[b])
                pltpu.sync_copy(gbuf.at[b], acc_sh.at[dstb.at[b]], add=True)

        pltpu.sync_copy(yc.at[pl.ds(s * rows_pw, rows_pw)],
                        acc_sh.at[pl.ds(s * rows_pw, rows_pw)])

        @pl.when(s == NSUB - 1)
        def _():
            pltpu.sync_copy(yc.at[pl.ds(NSUB * rows_pw, rem)],
                            acc_sh.at[pl.ds(NSUB * rows_pw, rem)])

        plsc.subcore_barrier()

        for b in range(nbuf):
            start_gather(b, b)

        @pl.loop(0, _cdiv(kmax, nbuf) * nbuf, step=nbuf)
        def _(k):
            for b in range(nbuf):
                finish_trip(k + b, b)
                start_gather(k + b + nbuf, b)

        plsc.subcore_barrier()

        pltpu.sync_copy(acc_sh.at[pl.ds(s * rows_pw, rows_pw)],
                        s_hbm.at[c, pl.ds(s * rows_pw, rows_pw)])

        @pl.when(s == NSUB - 1)
        def _():
            pltpu.sync_copy(acc_sh.at[pl.ds(NSUB * rows_pw, rem)],
                            s_hbm.at[c, pl.ds(NSUB * rows_pw, rem)])

    return k(y3, src, dst)


def _epilogue(s3, dinv, b, gamma, beta, rows_per_blk):
    n = s3.shape[1]
    half = s3.shape[2]
    dout = 2 * half
    grid = n // rows_per_blk
    bn_scale = np.float32(1.0 / np.sqrt(1.0 + 1e-5))

    def body(s_ref, dinv_ref, b_ref, g_ref, bt_ref, o_ref):
        dinv = dinv_ref[...]
        for h in range(2):
            sl = slice(h * half, (h + 1) * half)
            t = s_ref[h] * dinv + b_ref[:, sl]
            t = jnp.where(t >= 0, t, 0.01 * t)
            o_ref[:, sl] = t * (g_ref[:, sl] * bn_scale) + bt_ref[:, sl]

    return pl.pallas_call(
        body,
        grid=(grid,),
        in_specs=[
            pl.BlockSpec((NSC, rows_per_blk, half), lambda i: (0, i, 0)),
            pl.BlockSpec((rows_per_blk, 1), lambda i: (i, 0)),
            pl.BlockSpec((1, dout), lambda i: (0, 0)),
            pl.BlockSpec((1, dout), lambda i: (0, 0)),
            pl.BlockSpec((1, dout), lambda i: (0, 0)),
        ],
        out_specs=pl.BlockSpec((rows_per_blk, dout), lambda i: (i, 0)),
        out_shape=jax.ShapeDtypeStruct((n, dout), F32),
    )(s3, dinv, b, gamma, beta)


@jax.jit
def kernel(x, edge_index, W, b, gamma, beta):
    n = x.shape[0]
    src = edge_index[0]
    dst = edge_index[1]

    p0, p1 = _histogram(dst, n)
    y3, dinv = _matmul_scale(x, W, p0[:n].reshape(n, 1), p1[:n].reshape(n, 1),
                             rows_per_blk=1000)
    s3 = _gather_scatter(y3, src, dst, n)
    out = _epilogue(s3, dinv, b.reshape(1, -1), gamma.reshape(1, -1),
                    beta.reshape(1, -1), rows_per_blk=1000)
    return out


# speedup vs baseline: 18.0977x; 1.0051x over previous
import functools

import jax
import jax.numpy as jnp
import numpy as np
from jax import lax
from jax.experimental import pallas as pl
from jax.experimental.pallas import tpu as pltpu
from jax.experimental.pallas import tpu_sc as plsc

NSC = 2
NSUB = 16
LANES = 16
CH = 128

F32 = jnp.float32
I32 = jnp.int32


def _cdiv(a, b):
    return (a + b - 1) // b


def _histogram(dst, n):
    e = dst.shape[0]
    npad = _cdiv(n, NSUB * LANES) * NSUB * LANES
    zslice = npad // NSUB
    epc = e // NSC
    tchunks = epc // CH
    kmax = _cdiv(tchunks, NSUB)

    mesh = plsc.VectorSubcoreMesh(core_axis_name="c", subcore_axis_name="s")

    @functools.partial(
        pl.kernel,
        mesh=mesh,
        out_type=[
            jax.ShapeDtypeStruct((npad,), F32),
            jax.ShapeDtypeStruct((npad,), F32),
        ],
        scratch_types=[
            pltpu.VMEM_SHARED((npad,), F32),
            pltpu.VMEM((kmax, CH), I32),
            pltpu.VMEM((CH,), F32),
            pltpu.VMEM((zslice,), F32),
        ],
    )
    def k(dst_hbm, p0_hbm, p1_hbm, deg_sh, idx2, ones_v, zb):
        c = lax.axis_index("c")
        s = lax.axis_index("s")

        @pl.loop(0, zslice // LANES)
        def _(i):
            zb[pl.ds(i * LANES, LANES)] = jnp.zeros((LANES,), F32)

        pltpu.sync_copy(zb, deg_sh.at[pl.ds(s * zslice, zslice)])

        @pl.loop(0, CH // LANES)
        def _(i):
            ones_v[pl.ds(i * LANES, LANES)] = jnp.ones((LANES,), F32)

        plsc.subcore_barrier()

        @pl.loop(0, kmax)
        def _(k):
            ch = k * NSUB + s

            @pl.when(ch < tchunks)
            def _():
                base = c * epc + ch * CH
                pltpu.sync_copy(dst_hbm.at[pl.ds(base, CH)], idx2.at[k])
                pltpu.sync_copy(ones_v, deg_sh.at[idx2.at[k]], add=True)

        plsc.subcore_barrier()

        @pl.when(c == 0)
        def _():
            pltpu.sync_copy(deg_sh.at[pl.ds(s * zslice, zslice)],
                            p0_hbm.at[pl.ds(s * zslice, zslice)])

        @pl.when(c == 1)
        def _():
            pltpu.sync_copy(deg_sh.at[pl.ds(s * zslice, zslice)],
                            p1_hbm.at[pl.ds(s * zslice, zslice)])

    return k(dst)


def _matmul_scale(x, w, p0, p1, rows_per_blk):
    n, din = x.shape
    dout = w.shape[1]
    half = dout // 2
    grid = n // rows_per_blk

    def body(x_ref, w_ref, p0_ref, p1_ref, y_ref, dinv_ref):
        deg = p0_ref[...] + p1_ref[...] + 1.0
        dinv = lax.rsqrt(deg)
        dinv_ref[...] = dinv
        xw = jnp.dot(x_ref[...], w_ref[...], preferred_element_type=F32)
        y = xw * dinv
        y_ref[0] = y[:, :half]
        y_ref[1] = y[:, half:]

    return pl.pallas_call(
        body,
        grid=(grid,),
        in_specs=[
            pl.BlockSpec((rows_per_blk, din), lambda i: (i, 0)),
            pl.BlockSpec((din, dout), lambda i: (0, 0)),
            pl.BlockSpec((rows_per_blk, 1), lambda i: (i, 0)),
            pl.BlockSpec((rows_per_blk, 1), lambda i: (i, 0)),
        ],
        out_specs=[
            pl.BlockSpec((NSC, rows_per_blk, half), lambda i: (0, i, 0)),
            pl.BlockSpec((rows_per_blk, 1), lambda i: (i, 0)),
        ],
        out_shape=[
            jax.ShapeDtypeStruct((NSC, n, half), F32),
            jax.ShapeDtypeStruct((n, 1), F32),
        ],
    )(x, w, p0, p1)


def _gather_scatter(y3, src, dst, n):
    e = src.shape[0]
    half = y3.shape[2]
    tchunks = e // CH
    kmax = _cdiv(tchunks, NSUB)
    rows_pw = (n // NSUB) // 8 * 8
    rem = n - (NSUB - 1) * rows_pw - rows_pw

    mesh = plsc.VectorSubcoreMesh(core_axis_name="c", subcore_axis_name="s")

    nbuf = 3

    @functools.partial(
        pl.kernel,
        mesh=mesh,
        out_type=jax.ShapeDtypeStruct((NSC, n, half), F32),
        scratch_types=[
            pltpu.VMEM_SHARED((n, half), F32),
            pltpu.VMEM((nbuf, CH), I32),
            pltpu.VMEM((nbuf, CH), I32),
            pltpu.VMEM((nbuf, CH, half), F32),
            pltpu.SemaphoreType.DMA,
            pltpu.SemaphoreType.DMA,
            pltpu.SemaphoreType.DMA,
        ],
    )
    def k(y_hbm, src_hbm, dst_hbm, s_hbm, acc_sh, srcb, dstb, gbuf,
          sem0, sem1, sem2):
        c = lax.axis_index("c")
        s = lax.axis_index("s")
        yc = y_hbm.at[c]
        sems = (sem0, sem1, sem2)

        def start_gather(trip, b):
            @pl.when(trip * NSUB + s < tchunks)
            def _():
                base = (trip * NSUB + s) * CH
                pltpu.sync_copy(src_hbm.at[pl.ds(base, CH)], srcb.at[b])
                pltpu.async_copy(yc.at[srcb.at[b]], gbuf.at[b], sems[b])
                pltpu.sync_copy(dst_hbm.at[pl.ds(base, CH)], dstb.at[b])

        def finish_trip(trip, b):
            @pl.when(trip * NSUB + s < tchunks)
            def _():
                pltpu.make_async_copy(yc.at[srcb.at[b]], gbuf.at[b],
                                      sems[b]).wait()
                pltpu.sync_copy(gbuf.at[b], acc_sh.at[dstb.at[b]], add=True)

        pltpu.sync_copy(yc.at[pl.ds(s * rows_pw, rows_pw)],
                        acc_sh.at[pl.ds(s * rows_pw, rows_pw)])

        @pl.when(s == NSUB - 1)
        def _():
            pltpu.sync_copy(yc.at[pl.ds(NSUB * rows_pw, rem)],
                            acc_sh.at[pl.ds(NSUB * rows_pw, rem)])

        plsc.subcore_barrier()

        for b in range(nbuf):
            start_gather(b, b)

        @pl.loop(0, _cdiv(kmax, nbuf) * nbuf, step=nbuf)
        def _(k):
            for b in range(nbuf):
                finish_trip(k + b, b)
                start_gather(k + b + nbuf, b)

        plsc.subcore_barrier()

        pltpu.sync_copy(acc_sh.at[pl.ds(s * rows_pw, rows_pw)],
                        s_hbm.at[c, pl.ds(s * rows_pw, rows_pw)])

        @pl.when(s == NSUB - 1)
        def _():
            pltpu.sync_copy(acc_sh.at[pl.ds(NSUB * rows_pw, rem)],
                            s_hbm.at[c, pl.ds(NSUB * rows_pw, rem)])

    return k(y3, src, dst)


def _epilogue(s3, dinv, b, gamma, beta, rows_per_blk):
    n = s3.shape[1]
    half = s3.shape[2]
    dout = 2 * half
    grid = n // rows_per_blk
    bn_scale = np.float32(1.0 / np.sqrt(1.0 + 1e-5))

    def body(s_ref, dinv_ref, b_ref, g_ref, bt_ref, o_ref):
        dinv = dinv_ref[...]
        for h in range(2):
            sl = slice(h * half, (h + 1) * half)
            t = s_ref[h] * dinv + b_ref[:, sl]
            t = jnp.where(t >= 0, t, 0.01 * t)
            o_ref[:, sl] = t * (g_ref[:, sl] * bn_scale) + bt_ref[:, sl]

    return pl.pallas_call(
        body,
        grid=(grid,),
        in_specs=[
            pl.BlockSpec((NSC, rows_per_blk, half), lambda i: (0, i, 0)),
            pl.BlockSpec((rows_per_blk, 1), lambda i: (i, 0)),
            pl.BlockSpec((1, dout), lambda i: (0, 0)),
            pl.BlockSpec((1, dout), lambda i: (0, 0)),
            pl.BlockSpec((1, dout), lambda i: (0, 0)),
        ],
        out_specs=pl.BlockSpec((rows_per_blk, dout), lambda i: (i, 0)),
        out_shape=jax.ShapeDtypeStruct((n, dout), F32),
    )(s3, dinv, b, gamma, beta)


@jax.jit
def kernel(x, edge_index, W, b, gamma, beta):
    n = x.shape[0]
    src = edge_index[0]
    dst = edge_index[1]

    p0, p1 = _histogram(dst, n)
    y3, dinv = _matmul_scale(x, W, p0[:n].reshape(n, 1), p1[:n].reshape(n, 1),
                             rows_per_blk=1000)
    s3 = _gather_scatter(y3, src, dst, n)
    out = _epilogue(s3, dinv, b.reshape(1, -1), gamma.reshape(1, -1),
                    beta.reshape(1, -1), rows_per_blk=1000)
    return out
